# Initial kernel scaffold; baseline (speedup 1.0000x reference)
import jax
import jax.numpy as jnp
from jax.experimental import pallas as pl


def kernel(x, table):
    raise NotImplementedError("write your pallas kernel here")


# speedup vs baseline: 5.7871x; 5.7871x over previous
import functools

import jax
import jax.numpy as jnp
from jax import lax
from jax.experimental import pallas as pl
from jax.experimental.pallas import tpu as pltpu
from jax.experimental.pallas import tpu_sc as plsc

D_MODEL = 128
NUM_CORES = 2
NUM_SUBCORES = 16
NUM_WORKERS = NUM_CORES * NUM_SUBCORES
CHUNK = 128


def _emb_body(x_hbm, table_hbm, out_hbm, idx_v, rows_v, sem):
    n_chunks = x_hbm.shape[1]
    wid = lax.axis_index("s") * NUM_CORES + lax.axis_index("c")
    base = wid * n_chunks * CHUNK
    pltpu.sync_copy(x_hbm.at[wid], idx_v)

    def chunk_body(c, _):
        pltpu.async_copy(table_hbm.at[idx_v.at[c]], rows_v, sem).wait()
        pltpu.sync_copy(rows_v, out_hbm.at[pl.ds(base + c * CHUNK, CHUNK)])
        return 0

    lax.fori_loop(0, n_chunks, chunk_body, 0)


def kernel(x, table):
    b, t = x.shape
    total = b * t
    assert total % (NUM_WORKERS * CHUNK) == 0
    n_chunks = total // (NUM_WORKERS * CHUNK)
    x_blocks = x.reshape(NUM_WORKERS, n_chunks, CHUNK)

    emb = functools.partial(
        pl.kernel,
        out_type=jax.ShapeDtypeStruct((total, D_MODEL), jnp.float32),
        mesh=plsc.VectorSubcoreMesh(core_axis_name="c", subcore_axis_name="s"),
        scratch_types=[
            pltpu.VMEM((n_chunks, CHUNK), jnp.int32),
            pltpu.VMEM((CHUNK, D_MODEL), jnp.float32),
            pltpu.SemaphoreType.DMA,
        ],
    )(_emb_body)

    out = emb(x_blocks, table)
    return out.reshape(b, t, D_MODEL)


# speedup vs baseline: 8.0770x; 1.3957x over previous
import functools

import jax
import jax.numpy as jnp
from jax import lax
from jax.experimental import pallas as pl
from jax.experimental.pallas import tpu as pltpu
from jax.experimental.pallas import tpu_sc as plsc

D_MODEL = 128
NUM_CORES = 2
NUM_SUBCORES = 16
NUM_WORKERS = NUM_CORES * NUM_SUBCORES
CHUNK = 128
NBUF = 5


def _emb_body(x_hbm, table_hbm, out_hbm, idx_v, rows_v, gsem, wsem):
    n_chunks = x_hbm.shape[1]
    wid = lax.axis_index("s") * NUM_CORES + lax.axis_index("c")
    base = wid * n_chunks * CHUNK
    pltpu.sync_copy(x_hbm.at[wid], idx_v)

    def gather(c, b):
        pltpu.async_copy(table_hbm.at[idx_v.at[c]], rows_v.at[b], gsem.at[b])

    for b in range(NBUF):
        gather(b, b)

    def group_body(g, _):
        for b in range(NBUF):
            c = g * NBUF + b
            pltpu.make_async_copy(
                table_hbm.at[idx_v.at[c]], rows_v.at[b], gsem.at[b]
            ).wait()
            out_slab = out_hbm.at[pl.ds(base + c * CHUNK, CHUNK)]
            pltpu.async_copy(rows_v.at[b], out_slab, wsem.at[b])
            pltpu.make_async_copy(rows_v.at[b], out_slab, wsem.at[b]).wait()

            @pl.when(c + NBUF < n_chunks)
            def _():
                gather(c + NBUF, b)

        return 0

    lax.fori_loop(0, n_chunks // NBUF, group_body, 0)


def kernel(x, table):
    b, t = x.shape
    total = b * t
    assert total % (NUM_WORKERS * CHUNK * NBUF) == 0
    n_chunks = total // (NUM_WORKERS * CHUNK)
    x_blocks = x.reshape(NUM_WORKERS, n_chunks, CHUNK)

    emb = functools.partial(
        pl.kernel,
        out_type=jax.ShapeDtypeStruct((total, D_MODEL), jnp.float32),
        mesh=plsc.VectorSubcoreMesh(core_axis_name="c", subcore_axis_name="s"),
        scratch_types=[
            pltpu.VMEM((n_chunks, CHUNK), jnp.int32),
            pltpu.VMEM((NBUF, CHUNK, D_MODEL), jnp.float32),
            pltpu.SemaphoreType.DMA((NBUF,)),
            pltpu.SemaphoreType.DMA((NBUF,)),
        ],
    )(_emb_body)

    out = emb(x_blocks, table)
    return out.reshape(b, t, D_MODEL)
